# baseline (speedup 1.0000x reference)
import functools

import jax
import jax.numpy as jnp
from jax import lax
from jax.experimental import pallas as pl
from jax.experimental.pallas import tpu as pltpu
from jax.experimental.pallas import tpu_sc as plsc

N = 10000
E = 320000
F_IN = 128
H = 64
EPS = 1e-5

NC = 2
NS = 16
NW = NC * NS
CHUNK = 128
EPW = ((E + NW - 1) // NW + 2 * CHUNK - 1) // (2 * CHUNK) * (2 * CHUNK)
E_PAD = EPW * NW
N_PAD = 10240
SPT = N_PAD // NW
PCAP = EPW + 2 * CHUNK
PTOT = NW * PCAP + CHUNK
BCOLS = 48

_HI = lax.Precision.HIGHEST


def _sc_mesh():
    return plsc.VectorSubcoreMesh(core_axis_name="c", subcore_axis_name="s",
                                  num_cores=NC, num_subcores=NS)


def _sget(ref, pos):
    v = plsc.load_gather(ref, [jnp.broadcast_to(pos, (16,))])
    return v[0]


def _prep_body(src_hbm, dst_hbm, ew_hbm, psrc_hbm, pdst_hbm, pew_hbm, bnd_hbm,
               sall, dall, eall, ps, pd, pe, cnt, obuf, bbuf):
    c = lax.axis_index("c")
    s = lax.axis_index("s")
    i32 = jnp.int32
    wid = c * i32(NS) + s
    woff = wid * i32(EPW)
    iota = lax.iota(jnp.int32, 16)
    zi = jnp.zeros((16,), jnp.int32)
    zf = jnp.zeros((16,), jnp.float32)
    ones = jnp.ones((16,), jnp.int32)

    pltpu.sync_copy(src_hbm.at[pl.ds(woff, EPW)], sall)
    pltpu.sync_copy(dst_hbm.at[pl.ds(woff, EPW)], dall)
    pltpu.sync_copy(ew_hbm.at[pl.ds(woff, EPW)], eall)

    for t in range(512 // 16):
        cnt[pl.ds(t * 16, 16)] = zi

    def _zero(i, carry):
        ps[pl.ds(i * 16, 16)] = zi
        pd[pl.ds(i * 16, 16)] = zi
        pe[pl.ds(i * 16, 16)] = zf
        return carry

    lax.fori_loop(i32(0), i32(PCAP // 16), _zero, i32(0))

    def _hist(g, carry):
        d16 = dall[pl.ds(g * i32(16), 16)]
        b16 = d16 // i32(SPT)
        plsc.addupdate_scatter(cnt, [b16 * i32(16) + iota], ones)
        return carry

    lax.fori_loop(i32(0), i32(EPW // 16), _hist, i32(0))

    run = i32(0)
    for b in range(NW):
        run = ((run + i32(7)) // i32(8)) * i32(8)
        cv = cnt[pl.ds(b * 16, 16)]
        ex = plsc.cumsum(cv) - cv
        obuf[pl.ds(b * 16, 16)] = ex + jnp.broadcast_to(run, (16,))
        run = run + jnp.sum(cv, dtype=jnp.int32)

    for t in range(2):
        bidx = (iota + i32(t * 16)) * i32(16)
        bbuf[pl.ds(t * 16, 16)] = plsc.load_gather(obuf, [bidx])
    bbuf[pl.ds(32, 16)] = jnp.broadcast_to(run, (16,))

    def _scat(g, carry):
        gb = g * i32(16)
        d16 = dall[pl.ds(gb, 16)]
        s16 = sall[pl.ds(gb, 16)]
        e16 = eall[pl.ds(gb, 16)]
        idx = (d16 // i32(SPT)) * i32(16) + iota
        pos = plsc.load_gather(obuf, [idx])
        plsc.store_scatter(obuf, [idx], pos + i32(1))
        plsc.store_scatter(ps, [pos], s16)
        plsc.store_scatter(pd, [pos], d16)
        plsc.store_scatter(pe, [pos], e16)
        return carry

    lax.fori_loop(i32(0), i32(EPW // 16), _scat, i32(0))

    pltpu.sync_copy(ps, psrc_hbm.at[pl.ds(wid * i32(PCAP), PCAP)])
    pltpu.sync_copy(pd, pdst_hbm.at[pl.ds(wid * i32(PCAP), PCAP)])
    pltpu.sync_copy(pe, pew_hbm.at[pl.ds(wid * i32(PCAP), PCAP)])
    pltpu.sync_copy(bbuf, bnd_hbm.at[pl.ds(wid * i32(BCOLS), BCOLS)])


@functools.cache
def _prep_sc():
    return pl.kernel(
        _prep_body,
        out_type=[jax.ShapeDtypeStruct((PTOT,), jnp.int32),
                  jax.ShapeDtypeStruct((PTOT,), jnp.int32),
                  jax.ShapeDtypeStruct((PTOT,), jnp.float32),
                  jax.ShapeDtypeStruct((NW * BCOLS,), jnp.int32)],
        mesh=_sc_mesh(),
        compiler_params=pltpu.CompilerParams(use_tc_tiling_on_sc=False,
                                             needs_layout_passes=False),
        scratch_types=[
            pltpu.VMEM((EPW,), jnp.int32),
            pltpu.VMEM((EPW,), jnp.int32),
            pltpu.VMEM((EPW,), jnp.float32),
            pltpu.VMEM((PCAP,), jnp.int32),
            pltpu.VMEM((PCAP,), jnp.int32),
            pltpu.VMEM((PCAP,), jnp.float32),
            pltpu.VMEM((512,), jnp.int32),
            pltpu.VMEM((512,), jnp.int32),
            pltpu.VMEM((BCOLS,), jnp.int32),
        ],
    )


def _deg_body(pdst_hbm, pew_hbm, bnd_hbm, out_hbm, ball, dbuf, ebuf, dacc):
    c = lax.axis_index("c")
    s = lax.axis_index("s")
    i32 = jnp.int32
    wid = c * i32(NS) + s
    iota = lax.iota(jnp.int32, 16)
    zf = jnp.zeros((16,), jnp.float32)

    pltpu.sync_copy(bnd_hbm, ball)

    def _z(i, carry):
        dacc[pl.ds(i * 16, 16)] = zf
        return carry

    lax.fori_loop(i32(0), i32(SPT), _z, i32(0))

    def _seg(sw, carry):
        start = _sget(ball, sw * i32(BCOLS) + wid)
        end = _sget(ball, sw * i32(BCOLS) + wid + i32(1))
        base = sw * i32(PCAP)
        ends = jnp.broadcast_to(end, (16,))
        nch = (end - start + i32(CHUNK - 1)) // i32(CHUNK)

        def _ch(kc, ci):
            off = start + kc * i32(CHUNK)
            goff = pl.multiple_of(base + off, 8)
            pltpu.sync_copy(pdst_hbm.at[pl.ds(goff, CHUNK)], dbuf)
            pltpu.sync_copy(pew_hbm.at[pl.ds(goff, CHUNK)], ebuf)
            for g in range(CHUNK // 16):
                lpos = jnp.broadcast_to(off + i32(g * 16), (16,)) + iota
                m = lpos < ends
                e16 = jnp.where(m, ebuf[pl.ds(g * 16, 16)], zf)
                d16 = dbuf[pl.ds(g * 16, 16)]
                dl = jnp.minimum(jnp.maximum(d16 - wid * i32(SPT), i32(0)),
                                 i32(SPT - 1))
                plsc.addupdate_scatter(dacc, [dl * i32(16) + iota], e16)
            return ci

        lax.fori_loop(i32(0), nch, _ch, i32(0))
        return carry

    lax.fori_loop(i32(0), i32(NW), _seg, i32(0))
    pltpu.sync_copy(dacc, out_hbm.at[pl.ds(wid * i32(SPT * 16), SPT * 16)])


@functools.cache
def _deg_sc():
    return pl.kernel(
        _deg_body,
        out_type=jax.ShapeDtypeStruct((N_PAD * 16,), jnp.float32),
        mesh=_sc_mesh(),
        compiler_params=pltpu.CompilerParams(use_tc_tiling_on_sc=False,
                                             needs_layout_passes=False),
        scratch_types=[
            pltpu.VMEM((NW * BCOLS,), jnp.int32),
            pltpu.VMEM((CHUNK,), jnp.int32),
            pltpu.VMEM((CHUNK,), jnp.float32),
            pltpu.VMEM((SPT * 16,), jnp.float32),
        ],
    )


def _conv_body(xw_hbm, psrc_hbm, pdst_hbm, pew_hbm, bnd_hbm, out_hbm,
               ball, rows, sbuf, dbuf, ebuf, acc, gsem):
    c = lax.axis_index("c")
    s = lax.axis_index("s")
    i32 = jnp.int32
    wid = c * i32(NS) + s
    iota = lax.iota(jnp.int32, 16)
    zf = jnp.zeros((16,), jnp.float32)

    pltpu.sync_copy(bnd_hbm, ball)

    def _z(i, carry):
        for q in range(H // 16):
            acc[i, pl.ds(q * 16, 16)] = zf
        return carry

    lax.fori_loop(i32(0), i32(SPT), _z, i32(0))

    def _seg(sw, carry):
        start = _sget(ball, sw * i32(BCOLS) + wid)
        end = _sget(ball, sw * i32(BCOLS) + wid + i32(1))
        base = sw * i32(PCAP)
        ends = jnp.broadcast_to(end, (16,))
        nch = (end - start + i32(CHUNK - 1)) // i32(CHUNK)

        def _ch(kc, ci):
            off = start + kc * i32(CHUNK)
            goff = pl.multiple_of(base + off, 8)
            pltpu.sync_copy(psrc_hbm.at[pl.ds(goff, CHUNK)], sbuf)
            for g in range(CHUNK // 16):
                sv = sbuf[pl.ds(g * 16, 16)]
                sbuf[pl.ds(g * 16, 16)] = jnp.minimum(
                    jnp.maximum(sv, i32(0)), i32(N - 1))
            pltpu.sync_copy(pdst_hbm.at[pl.ds(goff, CHUNK)], dbuf)
            pltpu.sync_copy(pew_hbm.at[pl.ds(goff, CHUNK)], ebuf)
            pltpu.async_copy(xw_hbm.at[sbuf], rows, gsem).wait()
            for g in range(CHUNK // 16):
                lpos = jnp.broadcast_to(off + i32(g * 16), (16,)) + iota
                m = lpos < ends
                e16 = jnp.where(m, ebuf[pl.ds(g * 16, 16)], zf)
                dl16 = jnp.minimum(
                    jnp.maximum(dbuf[pl.ds(g * 16, 16)] - wid * i32(SPT),
                                i32(0)), i32(SPT - 1))
                for j in range(16):
                    r = dl16[j]
                    sj = jnp.broadcast_to(e16[j], (16,))
                    er = g * 16 + j
                    for q in range(H // 16):
                        acc[r, pl.ds(q * 16, 16)] = (
                            acc[r, pl.ds(q * 16, 16)]
                            + rows[er, pl.ds(q * 16, 16)] * sj)
            return ci

        lax.fori_loop(i32(0), nch, _ch, i32(0))
        return carry

    lax.fori_loop(i32(0), i32(NW), _seg, i32(0))
    pltpu.sync_copy(acc, out_hbm.at[pl.ds(wid * i32(SPT), SPT)])


@functools.cache
def _conv_sc():
    return pl.kernel(
        _conv_body,
        out_type=jax.ShapeDtypeStruct((N_PAD, H), jnp.float32),
        mesh=_sc_mesh(),
        compiler_params=pltpu.CompilerParams(use_tc_tiling_on_sc=False,
                                             needs_layout_passes=False),
        scratch_types=[
            pltpu.VMEM((NW * BCOLS,), jnp.int32),
            pltpu.VMEM((CHUNK, H), jnp.float32),
            pltpu.VMEM((CHUNK,), jnp.int32),
            pltpu.VMEM((CHUNK,), jnp.int32),
            pltpu.VMEM((CHUNK,), jnp.float32),
            pltpu.VMEM((SPT, H), jnp.float32),
            pltpu.SemaphoreType.DMA,
        ],
    )


def _tca_body(degp, x, w1, dis_o, xw1s_o):
    d = jnp.sum(degp[...][:N], axis=1, keepdims=True) + 1.0
    dis = 1.0 / jnp.sqrt(d)
    dis_o[...] = dis
    xw = jnp.dot(x[...], w1[...], precision=_HI,
                 preferred_element_type=jnp.float32)
    xw1s_o[...] = xw * dis


def _bn_in(pre, g, be):
    z = jnp.maximum(pre, 0.0)
    m = jnp.mean(z, axis=0, keepdims=True)
    v = jnp.mean((z - m) ** 2, axis=0, keepdims=True)
    return (z - m) / jnp.sqrt(v + EPS) * g + be


def _tcb_body(acc, xw1s, dis, b1, g1, be1, w2, x1_o, xw2s_o):
    dis_v = dis[...]
    pre = dis_v * (acc[...][:N] + xw1s[...]) + b1[...]
    x1 = _bn_in(pre, g1[...], be1[...])
    x1_o[...] = x1
    xw2s_o[...] = jnp.dot(x1, w2[...], precision=_HI,
                          preferred_element_type=jnp.float32) * dis_v


def _tcc1_body(acc, xw2s, dis, b2, g2, be2, x2_o):
    dis_v = dis[...]
    pre = dis_v * (acc[...][:N] + xw2s[...]) + b2[...]
    x2_o[...] = _bn_in(pre, g2[...], be2[...])


def _tcc2_body(x1, x2, x,
               wih1t, bih1, bhh1, wih2t, bih2, bhh2, linw, linb, out_o):
    xc = jnp.concatenate([x1[...], x2[...]], axis=1)
    gates = jnp.dot(xc, wih1t[...], precision=_HI,
                    preferred_element_type=jnp.float32) + bih1[...] + bhh1[...]
    i1 = jax.nn.sigmoid(gates[:, 0:H])
    g1g = jnp.tanh(gates[:, 2 * H:3 * H])
    o1 = jax.nn.sigmoid(gates[:, 3 * H:4 * H])
    h1 = o1 * jnp.tanh(i1 * g1g)
    gates2 = jnp.dot(h1, wih2t[...], precision=_HI,
                     preferred_element_type=jnp.float32) + bih2[...] + bhh2[...]
    i2 = jax.nn.sigmoid(gates2[:, 0:H])
    g2g = jnp.tanh(gates2[:, 2 * H:3 * H])
    o2 = jax.nn.sigmoid(gates2[:, 3 * H:4 * H])
    h2 = o2 * jnp.tanh(i2 * g2g)
    hcat = jnp.concatenate([jnp.maximum(h1, 0.0), jnp.maximum(h2, 0.0),
                            jnp.maximum(x[...], 0.0)], axis=1)
    out_o[...] = jnp.dot(hcat, linw[...], precision=_HI,
                         preferred_element_type=jnp.float32) + linb[...]


def _tca(degp, x, w1):
    return pl.pallas_call(
        _tca_body,
        out_shape=[jax.ShapeDtypeStruct((N, 1), jnp.float32),
                   jax.ShapeDtypeStruct((N, H), jnp.float32)],
    )(degp, x, w1)


def _tcb(acc, xw1s, dis, b1, g1, be1, w2):
    return pl.pallas_call(
        _tcb_body,
        out_shape=[jax.ShapeDtypeStruct((N, H), jnp.float32),
                   jax.ShapeDtypeStruct((N, H), jnp.float32)],
    )(acc, xw1s, dis, b1, g1, be1, w2)


def _tcc1(acc, xw2s, dis, b2, g2, be2):
    return pl.pallas_call(
        _tcc1_body,
        out_shape=jax.ShapeDtypeStruct((N, H), jnp.float32),
    )(acc, xw2s, dis, b2, g2, be2)


_RB = 2000


def _tcc2(x1, x2, x, wih1t, bih1, bhh1, wih2t, bih2, bhh2, linw, linb):
    row = lambda w: pl.BlockSpec((_RB, w), lambda i: (i, i * 0))
    full = lambda a, b: pl.BlockSpec((a, b), lambda i: (i * 0, i * 0))
    return pl.pallas_call(
        _tcc2_body,
        grid=(N // _RB,),
        in_specs=[row(H), row(H), row(F_IN),
                  full(2 * H, 4 * H), full(1, 4 * H), full(1, 4 * H),
                  full(H, 4 * H), full(1, 4 * H), full(1, 4 * H),
                  full(2 * H + F_IN, 1), full(1, 1)],
        out_specs=row(1),
        out_shape=jax.ShapeDtypeStruct((N, 1), jnp.float32),
    )(x1, x2, x, wih1t, bih1, bhh1, wih2t, bih2, bhh2, linw, linb)


def kernel(x, edge_index, edge_weight, W1, b1, W2, b2, g1, be1, g2, be2,
           Wih1, Whh1, bih1, bhh1, Wih2, Whh2, bih2, bhh2, linW, linb):
    src = edge_index[0].astype(jnp.int32)
    dst = edge_index[1].astype(jnp.int32)
    ew = edge_weight.astype(jnp.float32)
    pad = E_PAD - E
    if pad:
        zi = jnp.zeros((pad,), jnp.int32)
        src = jnp.concatenate([src, zi])
        dst = jnp.concatenate([dst, zi])
        ew = jnp.concatenate([ew, jnp.zeros((pad,), jnp.float32)])
    x = x.astype(jnp.float32)

    psrc, pdst, pew, bnd = _prep_sc()(src, dst, ew)
    degp = _deg_sc()(pdst, pew, bnd).reshape(N_PAD, 16)
    dis, xw1s = _tca(degp, x, W1.astype(jnp.float32))
    acc1 = _conv_sc()(xw1s, psrc, pdst, pew, bnd)
    x1, xw2s = _tcb(acc1, xw1s, dis,
                    b1.reshape(1, H).astype(jnp.float32),
                    g1.reshape(1, H).astype(jnp.float32),
                    be1.reshape(1, H).astype(jnp.float32),
                    W2.astype(jnp.float32))
    acc2 = _conv_sc()(xw2s, psrc, pdst, pew, bnd)
    x2 = _tcc1(acc2, xw2s, dis,
               b2.reshape(1, H).astype(jnp.float32),
               g2.reshape(1, H).astype(jnp.float32),
               be2.reshape(1, H).astype(jnp.float32))
    out = _tcc2(x1, x2, x,
                Wih1.astype(jnp.float32).T,
                bih1.reshape(1, 4 * H).astype(jnp.float32),
                bhh1.reshape(1, 4 * H).astype(jnp.float32),
                Wih2.astype(jnp.float32).T,
                bih2.reshape(1, 4 * H).astype(jnp.float32),
                bhh2.reshape(1, 4 * H).astype(jnp.float32),
                linW.astype(jnp.float32), linb.reshape(1, 1).astype(jnp.float32))
    return out
